# baseline (speedup 1.0000x reference)
import functools

import jax
import jax.numpy as jnp
import numpy as np
from jax import lax
from jax.experimental import pallas as pl
from jax.experimental.pallas import tpu as pltpu
from jax.experimental.pallas import tpu_sc as plsc

_B = 128
_C = 2048
_T = 128
_FOLD_DIV = 8
_NC = 2
_NS = 16
_NW = _NC * _NS
_K = 128
_RI = 7
_LEAD = 3


def _shift_codes():
    rng = np.random.default_rng(0)
    perm = rng.permutation(_C)
    fold = _C // _FOLD_DIV
    codes = np.zeros((_C,), np.int32)
    codes[np.sort(perm[:fold])] = 1
    codes[np.sort(perm[fold:2 * fold])] = 2
    return codes


def _row_groups():
    codes = _shift_codes()
    rows = np.arange(_B * _C, dtype=np.int32).reshape(_B, _C)
    out = []
    for code in (1, 2, 0):
        ch = np.nonzero(codes == code)[0]
        r = rows[:, ch].reshape(-1)
        n = r.size
        assert n % (_NW * _K) == 0, (code, n)
        out.append(r.reshape(_NW, n // (_NW * _K), _K))
    return out


_IDX_FWD, _IDX_BWD, _IDX_FIX = _row_groups()
_NCH_F = _IDX_FWD.shape[1]
_NCH_B = _IDX_BWD.shape[1]
_NCH_X = _IDX_FIX.shape[1]


def _chunk_order():
    shifted = [("f", j, 1) for j in range(_NCH_F)]
    shifted += [("b", j, 2) for j in range(_NCH_B)]
    fixed = [("x", j, 0) for j in range(_NCH_X)]
    order = []
    fi = si = 0
    while fi < len(fixed) or si < len(shifted):
        for _ in range(3):
            if fi < len(fixed):
                order.append(fixed[fi])
                fi += 1
        if si < len(shifted):
            order.append(shifted[si])
            si += 1
    return order


_CHUNKS = _chunk_order()


def _lane_perm(v, idx):
    return lax.gather(
        v, idx[:, None],
        dimension_numbers=lax.GatherDimensionNumbers(
            offset_dims=(), collapsed_slice_dims=(0,), start_index_map=(0,)),
        slice_sizes=(1,),
        mode=lax.GatherScatterMode.PROMISE_IN_BOUNDS)


def _shift_chunk(buf, code):
    lane = lax.iota(jnp.int32, 16)
    if code == 1:
        bidx = jnp.minimum(lane + 1, 15)
    else:
        bidx = jnp.maximum(lane - 1, 0)

    def body(r, carry):
        if code == 1:
            for i in range(7):
                buf[r, pl.ds(i * 16, 16)] = buf[r, pl.ds(i * 16 + 1, 16)]
            v = _lane_perm(buf[r, pl.ds(112, 16)], bidx)
            v = jnp.where(lane == 15, 0.0, v)
            buf[r, pl.ds(112, 16)] = v
        else:
            for i in range(7, 0, -1):
                buf[r, pl.ds(i * 16, 16)] = buf[r, pl.ds(i * 16 - 1, 16)]
            v = _lane_perm(buf[r, pl.ds(0, 16)], bidx)
            v = jnp.where(lane == 0, 0.0, v)
            buf[r, pl.ds(0, 16)] = v
        return carry

    lax.fori_loop(0, _K, body, 0)


_mesh = plsc.VectorSubcoreMesh(
    core_axis_name="c", subcore_axis_name="s",
    num_cores=_NC, num_subcores=_NS)


@functools.partial(
    pl.kernel,
    out_type=jax.ShapeDtypeStruct((_B * _C, _T), jnp.float32),
    mesh=_mesh,
    scratch_types=[
        pltpu.VMEM((_NCH_F, _K), jnp.int32),
        pltpu.VMEM((_NCH_B, _K), jnp.int32),
        pltpu.VMEM((_NCH_X, _K), jnp.int32),
        [pltpu.VMEM((_K, _T), jnp.float32) for _ in range(_RI)],
        [pltpu.SemaphoreType.DMA for _ in range(_RI)],
        [pltpu.SemaphoreType.DMA for _ in range(_RI)],
    ],
)
def _sc_shift(x_hbm, gf_hbm, gb_hbm, gx_hbm, o_hbm,
              vf, vb, vx, in_bufs, gsems, ssems):
    wid = lax.axis_index("s") * _NC + lax.axis_index("c")
    pltpu.sync_copy(gf_hbm.at[wid], vf)
    pltpu.sync_copy(gb_hbm.at[wid], vb)
    pltpu.sync_copy(gx_hbm.at[wid], vx)

    idx_refs = {"f": vf, "b": vb, "x": vx}
    n = len(_CHUNKS)
    gh = {}
    in_pending = [None] * _RI

    def start_gather(k):
        g, j, _ = _CHUNKS[k]
        s = k % _RI
        if in_pending[s] is not None:
            in_pending[s].wait()
            in_pending[s] = None
        gh[k] = pltpu.async_copy(
            x_hbm.at[idx_refs[g].at[j]], in_bufs[s], gsems[s])

    for k in range(min(_LEAD, n)):
        start_gather(k)
    for k in range(n):
        g, j, code = _CHUNKS[k]
        s = k % _RI
        gh.pop(k).wait()
        if code != 0:
            _shift_chunk(in_bufs[s], code)
        in_pending[s] = pltpu.async_copy(
            in_bufs[s], o_hbm.at[idx_refs[g].at[j]], ssems[s])
        if k + _LEAD < n:
            start_gather(k + _LEAD)
    for h in in_pending:
        if h is not None:
            h.wait()


@jax.jit
def _run(x2, gf, gb, gx):
    return _sc_shift(x2, gf, gb, gx)


def kernel(x):
    B, C, T = x.shape
    out = _run(x.reshape(B * C, T),
               jnp.asarray(_IDX_FWD), jnp.asarray(_IDX_BWD),
               jnp.asarray(_IDX_FIX))
    return out.reshape(B, C, T)


# speedup vs baseline: 1.0066x; 1.0066x over previous
import functools

import jax
import jax.numpy as jnp
import numpy as np
from jax import lax
from jax.experimental import pallas as pl
from jax.experimental.pallas import tpu as pltpu
from jax.experimental.pallas import tpu_sc as plsc

_B = 128
_C = 2048
_T = 128
_FOLD_DIV = 8
_NC = 2
_NS = 16
_NW = _NC * _NS
_K = 128
_RI = 7
_LEAD = 5


def _shift_codes():
    rng = np.random.default_rng(0)
    perm = rng.permutation(_C)
    fold = _C // _FOLD_DIV
    codes = np.zeros((_C,), np.int32)
    codes[np.sort(perm[:fold])] = 1
    codes[np.sort(perm[fold:2 * fold])] = 2
    return codes


def _row_groups():
    codes = _shift_codes()
    rows = np.arange(_B * _C, dtype=np.int32).reshape(_B, _C)
    out = []
    for code in (1, 2, 0):
        ch = np.nonzero(codes == code)[0]
        r = rows[:, ch].reshape(-1)
        n = r.size
        assert n % (_NW * _K) == 0, (code, n)
        out.append(r.reshape(_NW, n // (_NW * _K), _K))
    return out


_IDX_FWD, _IDX_BWD, _IDX_FIX = _row_groups()
_NCH_F = _IDX_FWD.shape[1]
_NCH_B = _IDX_BWD.shape[1]
_NCH_X = _IDX_FIX.shape[1]


def _chunk_order():
    shifted = [("f", j, 1) for j in range(_NCH_F)]
    shifted += [("b", j, 2) for j in range(_NCH_B)]
    fixed = [("x", j, 0) for j in range(_NCH_X)]
    order = []
    fi = si = 0
    while fi < len(fixed) or si < len(shifted):
        for _ in range(3):
            if fi < len(fixed):
                order.append(fixed[fi])
                fi += 1
        if si < len(shifted):
            order.append(shifted[si])
            si += 1
    return order


_CHUNKS = _chunk_order()


def _lane_perm(v, idx):
    return lax.gather(
        v, idx[:, None],
        dimension_numbers=lax.GatherDimensionNumbers(
            offset_dims=(), collapsed_slice_dims=(0,), start_index_map=(0,)),
        slice_sizes=(1,),
        mode=lax.GatherScatterMode.PROMISE_IN_BOUNDS)


def _shift_chunk(buf, code):
    lane = lax.iota(jnp.int32, 16)
    if code == 1:
        bidx = jnp.minimum(lane + 1, 15)
    else:
        bidx = jnp.maximum(lane - 1, 0)

    def body(r, carry):
        if code == 1:
            for i in range(7):
                buf[r, pl.ds(i * 16, 16)] = buf[r, pl.ds(i * 16 + 1, 16)]
            v = _lane_perm(buf[r, pl.ds(112, 16)], bidx)
            v = jnp.where(lane == 15, 0.0, v)
            buf[r, pl.ds(112, 16)] = v
        else:
            for i in range(7, 0, -1):
                buf[r, pl.ds(i * 16, 16)] = buf[r, pl.ds(i * 16 - 1, 16)]
            v = _lane_perm(buf[r, pl.ds(0, 16)], bidx)
            v = jnp.where(lane == 0, 0.0, v)
            buf[r, pl.ds(0, 16)] = v
        return carry

    lax.fori_loop(0, _K, body, 0)


_mesh = plsc.VectorSubcoreMesh(
    core_axis_name="c", subcore_axis_name="s",
    num_cores=_NC, num_subcores=_NS)


@functools.partial(
    pl.kernel,
    out_type=jax.ShapeDtypeStruct((_B * _C, _T), jnp.float32),
    mesh=_mesh,
    scratch_types=[
        pltpu.VMEM((_NCH_F, _K), jnp.int32),
        pltpu.VMEM((_NCH_B, _K), jnp.int32),
        pltpu.VMEM((_NCH_X, _K), jnp.int32),
        [pltpu.VMEM((_K, _T), jnp.float32) for _ in range(_RI)],
        [pltpu.SemaphoreType.DMA for _ in range(_RI)],
        [pltpu.SemaphoreType.DMA for _ in range(_RI)],
    ],
)
def _sc_shift(x_hbm, gf_hbm, gb_hbm, gx_hbm, o_hbm,
              vf, vb, vx, in_bufs, gsems, ssems):
    wid = lax.axis_index("s") * _NC + lax.axis_index("c")
    pltpu.sync_copy(gf_hbm.at[wid], vf)
    pltpu.sync_copy(gb_hbm.at[wid], vb)
    pltpu.sync_copy(gx_hbm.at[wid], vx)

    idx_refs = {"f": vf, "b": vb, "x": vx}
    n = len(_CHUNKS)
    gh = {}
    in_pending = [None] * _RI

    def start_gather(k):
        g, j, _ = _CHUNKS[k]
        s = k % _RI
        if in_pending[s] is not None:
            in_pending[s].wait()
            in_pending[s] = None
        gh[k] = pltpu.async_copy(
            x_hbm.at[idx_refs[g].at[j]], in_bufs[s], gsems[s])

    for k in range(min(_LEAD, n)):
        start_gather(k)
    for k in range(n):
        g, j, code = _CHUNKS[k]
        s = k % _RI
        gh.pop(k).wait()
        if code != 0:
            _shift_chunk(in_bufs[s], code)
        in_pending[s] = pltpu.async_copy(
            in_bufs[s], o_hbm.at[idx_refs[g].at[j]], ssems[s])
        if k + _LEAD < n:
            start_gather(k + _LEAD)
    for h in in_pending:
        if h is not None:
            h.wait()


@jax.jit
def _run(x2, gf, gb, gx):
    return _sc_shift(x2, gf, gb, gx)


def kernel(x):
    B, C, T = x.shape
    out = _run(x.reshape(B * C, T),
               jnp.asarray(_IDX_FWD), jnp.asarray(_IDX_BWD),
               jnp.asarray(_IDX_FIX))
    return out.reshape(B, C, T)
